# baseline (speedup 1.0000x reference)
import functools

import jax
import jax.numpy as jnp
from jax import lax
from jax.experimental import pallas as pl
from jax.experimental.pallas import tpu as pltpu
from jax.experimental.pallas import tpu_sc as plsc

D = 100000
N = 128
B = 16384
ARITY = 3

NC = 2
NS = 16
NW = NC * NS
QPW = B // NW
CH = 64
NCH = QPW // CH
LANES = 16
QPR = 128 // LANES
SCAT = CH * LANES // 128


@functools.partial(
    pl.kernel,
    mesh=plsc.VectorSubcoreMesh(core_axis_name="c", subcore_axis_name="s"),
    out_type=jax.ShapeDtypeStruct((B,), jnp.float32),
    scratch_types=[
        pltpu.VMEM((ARITY, NCH, CH), jnp.int32),
        pltpu.VMEM((CH, N), jnp.float32),
        pltpu.VMEM((CH, N), jnp.float32),
        pltpu.VMEM((CH, N), jnp.float32),
        pltpu.VMEM((CH, N), jnp.float32),
        pltpu.VMEM((CH, N), jnp.float32),
        pltpu.VMEM((CH, N), jnp.float32),
        pltpu.VMEM((CH, N), jnp.float32),
        pltpu.VMEM((CH, N), jnp.float32),
        pltpu.VMEM((CH, N), jnp.float32),
        pltpu.VMEM((CH, N), jnp.float32),
        pltpu.VMEM((CH, N), jnp.float32),
        pltpu.VMEM((CH, N), jnp.float32),
        pltpu.VMEM((NCH * SCAT, 128), jnp.float32),
        pltpu.VMEM((NCH * SCAT, 128), jnp.int32),
        pltpu.VMEM((QPW,), jnp.float32),
        pltpu.VMEM_SHARED((NS * QPW,), jnp.float32),
        pltpu.SemaphoreType.DMA,
        pltpu.SemaphoreType.DMA,
        pltpu.SemaphoreType.DMA,
        pltpu.SemaphoreType.DMA,
        pltpu.SemaphoreType.DMA,
    ],
)
def _rap_sc(idx_hbm, w_hbm, out_hbm, idx_v,
            s0a, s0b, s0c, s1a, s1b, s1c, s2a, s2b, s2c, s3a, s3b, s3c,
            pball, sidx, out_v, out_sh,
            sem0, sem1, sem2, sem3, psem):
    sub = lax.axis_index("s")
    wid = sub * NC + lax.axis_index("c")
    base = wid * QPW
    sbase = sub * QPW
    slots = ((s0a, s0b, s0c), (s1a, s1b, s1c),
             (s2a, s2b, s2c), (s3a, s3b, s3c))
    sems = (sem0, sem1, sem2, sem3)

    pltpu.sync_copy(idx_hbm.at[wid], idx_v)

    def fire(c, par):
        for a in range(ARITY):
            pltpu.async_copy(w_hbm.at[idx_v.at[a, c]], slots[par][a],
                             sems[par])

    def drain_gather(par):
        for a in range(ARITY):
            pltpu.make_async_copy(w_hbm.at[pl.ds(0, CH)], slots[par][a],
                                  sems[par]).wait()

    fire(0, 0)
    fire(1, 1)
    fire(2, 2)

    zero = jnp.zeros((LANES,), jnp.float32)

    def ibody(r, _):
        for k in range(QPR):
            sidx[r, pl.ds(k * LANES, LANES)] = jnp.full(
                (LANES,), sbase + r * QPR + k, jnp.int32)
        return 0

    lax.fori_loop(0, NCH * SCAT, ibody, 0)

    def zbody(g, _):
        out_v[pl.ds(g * LANES, LANES)] = zero
        return 0

    lax.fori_loop(0, QPW // LANES, zbody, 0)
    pltpu.sync_copy(out_v, out_sh.at[pl.ds(sbase, QPW)])

    def pair_body(p, _):
        for par in range(4):
            c = p * 4 + par

            @pl.when(c + 3 < NCH)
            def _():
                fire(c + 3, (par + 3) % 4)

            drain_gather(par)
            r0, r1, r2 = slots[par]

            def tree_partial(q, r0=r0, r1=r1, r2=r2):
                prods = [r0[q, pl.ds(j * LANES, LANES)]
                         * r1[q, pl.ds(j * LANES, LANES)]
                         * r2[q, pl.ds(j * LANES, LANES)]
                         for j in range(N // LANES)]
                while len(prods) > 1:
                    prods = [prods[i] + prods[i + 1]
                             for i in range(0, len(prods), 2)]
                return prods[0] * (1.0 / N)

            def qbody(i, _, c=c, tree_partial=tree_partial):
                for k in range(QPR):
                    pball[c * SCAT + i, pl.ds(k * LANES, LANES)] = (
                        tree_partial(i * QPR + k))
                return 0

            lax.fori_loop(0, SCAT, qbody, 0)

            for b in range(SCAT):
                pltpu.async_copy(pball.at[c * SCAT + b],
                                 out_sh.at[sidx.at[c * SCAT + b]],
                                 psem, add=True)
        return 0

    lax.fori_loop(0, NCH // 4, pair_body, 0)

    pltpu.make_async_copy(w_hbm.at[pl.ds(0, NCH * SCAT)], pball, psem).wait()
    pltpu.sync_copy(out_sh.at[pl.ds(sbase, QPW)], out_hbm.at[pl.ds(base, QPW)])


def kernel(q_t_idxs, W):
    idx = q_t_idxs.astype(jnp.int32)
    idx = idx.reshape(NW, NCH, CH, ARITY).transpose(0, 3, 1, 2)
    return _rap_sc(idx, W)


# speedup vs baseline: 1.0765x; 1.0765x over previous
import functools

import jax
import jax.numpy as jnp
from jax import lax
from jax.experimental import pallas as pl
from jax.experimental.pallas import tpu as pltpu
from jax.experimental.pallas import tpu_sc as plsc

D = 100000
N = 128
B = 16384
ARITY = 3

NC = 2
NS = 16
NW = NC * NS
QPW = B // NW
CH = 64
NCH = QPW // CH
LANES = 16
QPR = 128 // LANES
SCAT = CH * LANES // 128


@functools.partial(
    pl.kernel,
    mesh=plsc.VectorSubcoreMesh(core_axis_name="c", subcore_axis_name="s"),
    out_type=jax.ShapeDtypeStruct((B,), jnp.float32),
    scratch_types=[
        pltpu.VMEM((ARITY, NCH, CH), jnp.int32),
        pltpu.VMEM((CH, N), jnp.float32),
        pltpu.VMEM((CH, N), jnp.float32),
        pltpu.VMEM((CH, N), jnp.float32),
        pltpu.VMEM((CH, N), jnp.float32),
        pltpu.VMEM((CH, N), jnp.float32),
        pltpu.VMEM((CH, N), jnp.float32),
        pltpu.VMEM((NCH * SCAT, 128), jnp.float32),
        pltpu.VMEM((NCH * SCAT, 128), jnp.int32),
        pltpu.VMEM((QPW,), jnp.float32),
        pltpu.VMEM_SHARED((NS * QPW,), jnp.float32),
        pltpu.SemaphoreType.DMA,
        pltpu.SemaphoreType.DMA,
        pltpu.SemaphoreType.DMA,
    ],
)
def _rap_sc(idx_hbm, w_hbm, out_hbm, idx_v,
            s0a, s0b, s0c, s1a, s1b, s1c, pball, sidx, out_v, out_sh,
            sem0, sem1, psem):
    sub = lax.axis_index("s")
    wid = sub * NC + lax.axis_index("c")
    base = wid * QPW
    sbase = sub * QPW
    slots = ((s0a, s0b, s0c), (s1a, s1b, s1c))
    sems = (sem0, sem1)

    pltpu.sync_copy(idx_hbm.at[wid], idx_v)

    def fire(c, par):
        for a in range(ARITY):
            pltpu.async_copy(w_hbm.at[idx_v.at[a, c]], slots[par][a],
                             sems[par])

    def drain_gather(par):
        for a in range(ARITY):
            pltpu.make_async_copy(w_hbm.at[pl.ds(0, CH)], slots[par][a],
                                  sems[par]).wait()

    fire(0, 0)

    zero = jnp.zeros((LANES,), jnp.float32)

    def ibody(r, _):
        for k in range(QPR):
            sidx[r, pl.ds(k * LANES, LANES)] = jnp.full(
                (LANES,), sbase + r * QPR + k, jnp.int32)
        return 0

    lax.fori_loop(0, NCH * SCAT, ibody, 0)

    def zbody(g, _):
        out_v[pl.ds(g * LANES, LANES)] = zero
        return 0

    lax.fori_loop(0, QPW // LANES, zbody, 0)
    pltpu.sync_copy(out_v, out_sh.at[pl.ds(sbase, QPW)])

    def pair_body(p, _):
        for par in range(2):
            c = p * 2 + par

            @pl.when(c + 1 < NCH)
            def _():
                fire(c + 1, 1 - par)

            drain_gather(par)
            r0, r1, r2 = slots[par]

            def tree_partial(q, r0=r0, r1=r1, r2=r2):
                prods = [r0[q, pl.ds(j * LANES, LANES)]
                         * r1[q, pl.ds(j * LANES, LANES)]
                         * r2[q, pl.ds(j * LANES, LANES)]
                         for j in range(N // LANES)]
                while len(prods) > 1:
                    prods = [prods[i] + prods[i + 1]
                             for i in range(0, len(prods), 2)]
                return prods[0] * (1.0 / N)

            def qbody(i, _, c=c, tree_partial=tree_partial):
                for k in range(QPR):
                    pball[c * SCAT + i, pl.ds(k * LANES, LANES)] = (
                        tree_partial(i * QPR + k))
                return 0

            lax.fori_loop(0, SCAT, qbody, 0)

            for b in range(SCAT):
                pltpu.async_copy(pball.at[c * SCAT + b],
                                 out_sh.at[sidx.at[c * SCAT + b]],
                                 psem, add=True)
        return 0

    lax.fori_loop(0, NCH // 2, pair_body, 0)

    pltpu.make_async_copy(w_hbm.at[pl.ds(0, NCH * SCAT)], pball, psem).wait()
    pltpu.sync_copy(out_sh.at[pl.ds(sbase, QPW)], out_hbm.at[pl.ds(base, QPW)])


def kernel(q_t_idxs, W):
    idx = q_t_idxs.astype(jnp.int32)
    idx = idx.reshape(NW, NCH, CH, ARITY).transpose(0, 3, 1, 2)
    return _rap_sc(idx, W)
